# baseline (speedup 1.0000x reference)
import functools

import jax
import jax.numpy as jnp
from jax import lax
from jax.experimental import pallas as pl
from jax.experimental.pallas import tpu as pltpu
from jax.experimental.pallas import tpu_sc as plsc

V = 32000
SMOOTHING = 0.1
IGNORE = 0
CONF = 1.0 - SMOOTHING
SMOOTH = SMOOTHING / (V - 1)

N_ROWS = 4096
BR = 128
SC_ROWS = 1024
NT = N_ROWS - SC_ROWS
NW = 32
RPW = SC_ROWS // NW
LANES = 16


def _tc_body(ts_ref, tv_ref, p_ref, loss_ref, cnt_ref):
    i = pl.program_id(0)
    t = tv_ref[0, 0, :]

    W = 128
    C = V // W
    macc = p_ref[:, 0:W]
    qacc = p_ref[:, 0:W]
    for k in range(1, C):
        x = p_ref[:, k * W:(k + 1) * W]
        macc = jnp.maximum(macc, x)
        qacc = qacc + x
    m = jnp.max(macc, axis=1, keepdims=True)
    psum = jnp.sum(qacc, axis=1)

    sacc = jnp.exp(p_ref[:, 0:W] - m)
    for k in range(1, C):
        sacc = sacc + jnp.exp(p_ref[:, k * W:(k + 1) * W] - m)
    s = jnp.sum(sacc, axis=1)

    rows = []
    lane = jax.lax.broadcasted_iota(jnp.int32, (1, 128), 1)
    for r in range(BR):
        tr = ts_ref[0, 0, r]
        off = (tr // 128) * 128
        x = p_ref[pl.ds(r, 1), pl.ds(off, 128)]
        rows.append(jnp.where(lane == (tr - off), x, 0.0))
    pt = jnp.sum(jnp.concatenate(rows, axis=0), axis=1)

    lse = m[:, 0] + jnp.log(s)
    maskf = (t != IGNORE).astype(jnp.float32)
    loss = -(SMOOTH * (psum - V * lse) + (CONF - SMOOTH) * (pt - lse))

    @pl.when(i == 0)
    def _():
        loss_ref[0, 0] = 0.0
        cnt_ref[0, 0] = 0.0

    loss_ref[0, 0] += jnp.sum(loss * maskf)
    cnt_ref[0, 0] += jnp.sum(maskf)


def _tc_partial(p_full, t3):
    return pl.pallas_call(
        _tc_body,
        grid=(NT // BR,),
        in_specs=[
            pl.BlockSpec((1, 1, BR), lambda i: (i, 0, 0),
                         memory_space=pltpu.SMEM),
            pl.BlockSpec((1, 1, BR), lambda i: (i, 0, 0)),
            pl.BlockSpec((BR, V), lambda i: (i, 0)),
        ],
        out_specs=[
            pl.BlockSpec((1, 1), lambda i: (0, 0), memory_space=pltpu.SMEM),
            pl.BlockSpec((1, 1), lambda i: (0, 0), memory_space=pltpu.SMEM),
        ],
        out_shape=[
            jax.ShapeDtypeStruct((1, 1), jnp.float32),
            jax.ShapeDtypeStruct((1, 1), jnp.float32),
        ],
    )(t3, t3, p_full)


UNROLL = 8
CHUNK = UNROLL * LANES
N_ITER = V // CHUNK


def _sc_row_partials(rowbuf, tsel):
    neg_inf = jnp.full((LANES,), -3.0e38, jnp.float32)
    zeros = jnp.zeros((LANES,), jnp.float32)

    def pass1(k, carry):
        maccs, qaccs = carry
        base = k * CHUNK
        maccs = list(maccs)
        qaccs = list(qaccs)
        for u in range(UNROLL):
            v = rowbuf[pl.ds(base + u * LANES, LANES)]
            maccs[u] = jnp.maximum(maccs[u], v)
            qaccs[u] = qaccs[u] + v
        return tuple(maccs), tuple(qaccs)

    maccs, qaccs = lax.fori_loop(
        0, N_ITER, pass1,
        (tuple([neg_inf] * UNROLL), tuple([zeros] * UNROLL)))
    mvecs = list(maccs)
    qvecs = list(qaccs)
    while len(mvecs) > 1:
        mvecs = [jnp.maximum(mvecs[i], mvecs[i + 1])
                 for i in range(0, len(mvecs), 2)]
        qvecs = [qvecs[i] + qvecs[i + 1] for i in range(0, len(qvecs), 2)]
    m_vec = mvecs[0]
    q_vec = qvecs[0]

    def pass2(k, saccs):
        base = k * CHUNK
        saccs = list(saccs)
        for u in range(UNROLL):
            v = rowbuf[pl.ds(base + u * LANES, LANES)]
            saccs[u] = saccs[u] + jnp.exp(v - m_vec)
        return tuple(saccs)

    saccs = lax.fori_loop(0, N_ITER, pass2, tuple([zeros] * UNROLL))
    svecs = list(saccs)
    while len(svecs) > 1:
        svecs = [svecs[i] + svecs[i + 1] for i in range(0, len(svecs), 2)]
    s_vec = svecs[0]

    start = pl.multiple_of((tsel // LANES) * LANES, 8)
    vec = rowbuf[pl.ds(start, LANES)]
    lane_iota = lax.iota(jnp.int32, LANES)
    pt_vec = jnp.where(lane_iota == tsel - start, vec, 0.0)
    return m_vec, s_vec, q_vec, pt_vec


GRP = LANES * LANES


def _sc_partials(p_hbm_arr, t_hbm_arr):
    mesh = plsc.VectorSubcoreMesh(core_axis_name="c", subcore_axis_name="s")
    out_sds = jax.ShapeDtypeStruct((SC_ROWS * LANES,), jnp.float32)

    @functools.partial(
        pl.kernel, mesh=mesh,
        out_type=[out_sds, out_sds, out_sds, out_sds],
        scratch_types=[
            pltpu.VMEM((V,), jnp.float32),
            pltpu.VMEM((V,), jnp.float32),
            pltpu.VMEM((RPW,), jnp.int32),
            pltpu.VMEM((GRP,), jnp.float32),
            pltpu.VMEM((GRP,), jnp.float32),
            pltpu.VMEM((GRP,), jnp.float32),
            pltpu.VMEM((GRP,), jnp.float32),
            pltpu.SemaphoreType.DMA,
            pltpu.SemaphoreType.DMA,
        ],
    )
    def sc_kernel(p_hbm, t_hbm, m_out, s_out, q_out, pt_out,
                  buf0, buf1, tgt_v, mb, sb, qb, ptb, sem0, sem1):
        wid = lax.axis_index("s") * 2 + lax.axis_index("c")
        rbase = pl.multiple_of(wid * RPW, 8)
        gbase = pl.multiple_of(NT + rbase, 8)

        pltpu.sync_copy(t_hbm.at[pl.ds(gbase, RPW)], tgt_v)
        pltpu.make_async_copy(p_hbm.at[gbase], buf0, sem0).start()
        pltpu.make_async_copy(p_hbm.at[gbase + 1], buf1, sem1).start()

        bufs = (buf0, buf1)
        sems = (sem0, sem1)

        def group(g, _):
            g16 = pl.multiple_of(g * LANES, 8)
            t_chunk = tgt_v[pl.ds(g16, LANES)]
            for u in range(LANES):
                j = g16 + u
                buf = bufs[u % 2]
                sem = sems[u % 2]
                pltpu.make_async_copy(p_hbm.at[gbase + j], buf, sem).wait()
                tsel = t_chunk[u]
                m_vec, s_vec, q_vec, pt_vec = _sc_row_partials(buf, tsel)

                @pl.when(j + 2 < RPW)
                def _():
                    pltpu.make_async_copy(p_hbm.at[gbase + j + 2], buf, sem
                                          ).start()

                mb[pl.ds(u * LANES, LANES)] = m_vec
                sb[pl.ds(u * LANES, LANES)] = s_vec
                qb[pl.ds(u * LANES, LANES)] = q_vec
                ptb[pl.ds(u * LANES, LANES)] = pt_vec

            off = pl.multiple_of((rbase + g16) * LANES, 8)
            pltpu.sync_copy(mb, m_out.at[pl.ds(off, GRP)])
            pltpu.sync_copy(sb, s_out.at[pl.ds(off, GRP)])
            pltpu.sync_copy(qb, q_out.at[pl.ds(off, GRP)])
            pltpu.sync_copy(ptb, pt_out.at[pl.ds(off, GRP)])
            return 0

        lax.fori_loop(0, RPW // LANES, group, 0)

    return sc_kernel(p_hbm_arr, t_hbm_arr)


def _combine_body(t_ref, m_ref, s_ref, q_ref, pt_ref,
                  tls_ref, tcnt_ref, out_ref):
    m2 = m_ref[...]
    s2 = s_ref[...]
    m = jnp.max(m2, axis=1, keepdims=True)
    s = jnp.sum(s2 * jnp.exp(m2 - m), axis=1, keepdims=True)
    q = jnp.sum(q_ref[...], axis=1, keepdims=True)
    pt = jnp.sum(pt_ref[...], axis=1, keepdims=True)
    lse = m + jnp.log(s)
    maskf = (t_ref[...] != IGNORE).astype(jnp.float32)
    loss = -(SMOOTH * (q - V * lse) + (CONF - SMOOTH) * (pt - lse))
    loss_sum = jnp.sum(loss * maskf) + tls_ref[0, 0]
    cnt = jnp.sum(maskf) + tcnt_ref[0, 0]
    out_ref[0, 0] = loss_sum / cnt


def _combine(t_sc, m, s, q, pt, tc_ls, tc_cnt):
    shp = (SC_ROWS, LANES)
    tshp = (SC_ROWS, 1)
    out = pl.pallas_call(
        _combine_body,
        in_specs=[
            pl.BlockSpec(tshp, lambda: (0, 0)),
            pl.BlockSpec(shp, lambda: (0, 0)),
            pl.BlockSpec(shp, lambda: (0, 0)),
            pl.BlockSpec(shp, lambda: (0, 0)),
            pl.BlockSpec(shp, lambda: (0, 0)),
            pl.BlockSpec((1, 1), lambda: (0, 0), memory_space=pltpu.SMEM),
            pl.BlockSpec((1, 1), lambda: (0, 0), memory_space=pltpu.SMEM),
        ],
        out_specs=pl.BlockSpec((1, 1), lambda: (0, 0),
                               memory_space=pltpu.SMEM),
        out_shape=jax.ShapeDtypeStruct((1, 1), jnp.float32),
    )(t_sc.reshape(tshp), m.reshape(shp), s.reshape(shp), q.reshape(shp),
      pt.reshape(shp), tc_ls, tc_cnt)
    return out[0, 0]


def kernel(pred, target):
    p = pred.reshape(-1, V)
    t = target.reshape(-1).astype(jnp.int32)

    t3 = t[:NT].reshape(NT // BR, 1, BR)
    tc_ls, tc_cnt = _tc_partial(p, t3)
    m, s, q, pt = _sc_partials(p, t)
    return _combine(t[NT:], m, s, q, pt, tc_ls, tc_cnt)


# speedup vs baseline: 1.1604x; 1.1604x over previous
import jax
import jax.numpy as jnp
from jax.experimental import pallas as pl
from jax.experimental.pallas import tpu as pltpu

V = 32000
SMOOTHING = 0.1
IGNORE = 0
CONF = 1.0 - SMOOTHING
SMOOTH = SMOOTHING / (V - 1)

BR = 128
W = 256
C = V // W
K = 8


def _acc_reduce(op, slices):
    accs = list(slices[:K])
    for k in range(K, len(slices)):
        accs[k % K] = op(accs[k % K], slices[k])
    while len(accs) > 1:
        nxt = [op(accs[i], accs[i + 1]) for i in range(0, len(accs) - 1, 2)]
        if len(accs) % 2:
            nxt.append(accs[-1])
        accs = nxt
    return accs[0]


def _body(ts_ref, tv_ref, p_ref, loss_ref, cnt_ref):
    i = pl.program_id(0)
    t = tv_ref[0, 0, :]

    xs = [p_ref[:, k * W:(k + 1) * W] for k in range(C)]
    m_l = _acc_reduce(jnp.maximum, xs)
    m = jnp.max(m_l, axis=1, keepdims=True)
    psum = jnp.sum(_acc_reduce(jnp.add, xs), axis=1)

    es = [jnp.exp(p_ref[:, k * W:(k + 1) * W] - m) for k in range(C)]
    s = jnp.sum(_acc_reduce(jnp.add, es), axis=1)

    rows = []
    lane = jax.lax.broadcasted_iota(jnp.int32, (1, 128), 1)
    for r in range(BR):
        tr = ts_ref[0, 0, r]
        off = (tr // 128) * 128
        x = p_ref[pl.ds(r, 1), pl.ds(off, 128)]
        rows.append(jnp.where(lane == (tr - off), x, 0.0))
    pt = jnp.sum(jnp.concatenate(rows, axis=0), axis=1)

    lse = m[:, 0] + jnp.log(s)
    maskf = (t != IGNORE).astype(jnp.float32)
    loss = -(SMOOTH * (psum - V * lse) + (CONF - SMOOTH) * (pt - lse))

    @pl.when(i == 0)
    def _():
        loss_ref[0, 0] = 0.0
        cnt_ref[0, 0] = 0.0

    loss_ref[0, 0] += jnp.sum(loss * maskf)
    cnt_ref[0, 0] += jnp.sum(maskf)


def kernel(pred, target):
    p = pred.reshape(-1, V)
    n = p.shape[0]
    nb = n // BR
    t3 = target.reshape(nb, 1, BR).astype(jnp.int32)

    loss_sum, cnt = pl.pallas_call(
        _body,
        grid=(nb,),
        in_specs=[
            pl.BlockSpec((1, 1, BR), lambda i: (i, 0, 0),
                         memory_space=pltpu.SMEM),
            pl.BlockSpec((1, 1, BR), lambda i: (i, 0, 0)),
            pl.BlockSpec((BR, V), lambda i: (i, 0)),
        ],
        out_specs=[
            pl.BlockSpec((1, 1), lambda i: (0, 0), memory_space=pltpu.SMEM),
            pl.BlockSpec((1, 1), lambda i: (0, 0), memory_space=pltpu.SMEM),
        ],
        out_shape=[
            jax.ShapeDtypeStruct((1, 1), jnp.float32),
            jax.ShapeDtypeStruct((1, 1), jnp.float32),
        ],
    )(t3, t3, p)
    return loss_sum[0, 0] / cnt[0, 0]
